# Initial kernel scaffold; baseline (speedup 1.0000x reference)
import functools

import jax
import jax.numpy as jnp
from jax import lax
from jax.experimental import pallas as pl
from jax.experimental.pallas import tpu as pltpu
from jax.experimental.pallas import tpu_sc as plsc


def _make_pool(batch, hist, d, nc, ns):
    nw = nc * ns
    cb = 2
    ct = cb * hist
    chunks_per_w = batch // (cb * nw)
    bags_per_w = batch // nw
    half = chunks_per_w // 2
    assert batch % (cb * nw) == 0 and chunks_per_w % 2 == 0 and d % 16 == 0
    nd = d // 16

    mesh = plsc.VectorSubcoreMesh(core_axis_name="c", subcore_axis_name="s")

    @functools.partial(
        pl.kernel,
        mesh=mesh,
        out_type=jax.ShapeDtypeStruct((batch, d), jnp.float32),
        scratch_types=[
            pltpu.VMEM((chunks_per_w, ct), jnp.int32),
            pltpu.VMEM((ct, d), jnp.float32),
            pltpu.VMEM((ct, d), jnp.float32),
            pltpu.VMEM((bags_per_w, d), jnp.float32),
            pltpu.SemaphoreType.DMA,
            pltpu.SemaphoreType.DMA,
        ],
    )
    def pool(tok_hbm, table_hbm, out_hbm, idx_v, buf0, buf1, pooled_v,
             sem0, sem1):
        wid = lax.axis_index("s") * nc + lax.axis_index("c")
        pltpu.sync_copy(tok_hbm.at[pl.ds(wid * chunks_per_w, chunks_per_w)],
                        idx_v)

        def start(c, buf, sem):
            pltpu.make_async_copy(table_hbm.at[idx_v.at[c]], buf, sem).start()

        def wait(buf, sem):
            pltpu.make_async_copy(table_hbm.at[idx_v.at[0]], buf, sem).wait()

        def accum(buf, bag_base):
            for j in range(cb):
                accs = [buf[j * hist, pl.ds(k * 16, 16)] for k in range(nd)]
                for t in range(1, hist):
                    for k in range(nd):
                        accs[k] = accs[k] + buf[j * hist + t,
                                                pl.ds(k * 16, 16)]
                for k in range(nd):
                    pooled_v[bag_base + j, pl.ds(k * 16, 16)] = accs[k]

        start(0, buf0, sem0)

        def step(g, carry):
            start(2 * g + 1, buf1, sem1)
            wait(buf0, sem0)
            accum(buf0, 2 * cb * g)

            @pl.when(g < half - 1)
            def _():
                start(2 * g + 2, buf0, sem0)

            wait(buf1, sem1)
            accum(buf1, 2 * cb * g + cb)
            return carry

        lax.fori_loop(0, half, step, 0)
        pltpu.sync_copy(pooled_v,
                        out_hbm.at[pl.ds(wid * bags_per_w, bags_per_w)])

    return pool


def _fc_body(scale, pooled_ref, wt_ref, b_ref, out_ref):
    out_ref[...] = (
        jnp.dot(pooled_ref[...] * scale, wt_ref[...],
                preferred_element_type=jnp.float32)
        + b_ref[...]
    )


def kernel(concated_token_lists, offsets, emb_weight, fc_weight, fc_bias):
    total = concated_token_lists.shape[0]
    batch = offsets.shape[0]
    hist = total // batch
    d = emb_weight.shape[1]
    ncls = fc_weight.shape[0]

    info = plsc.get_sparse_core_info()
    nc, ns = info.num_cores, info.num_subcores

    cb = 2
    tok2d = concated_token_lists.astype(jnp.int32).reshape(
        batch // cb, hist * cb)
    pooled = _make_pool(batch, hist, d, nc, ns)(tok2d, emb_weight)

    wt = fc_weight.T.astype(jnp.float32)
    b2 = fc_bias.reshape(1, ncls).astype(jnp.float32)
    fc = pl.pallas_call(
        functools.partial(_fc_body, 1.0 / float(hist)),
        out_shape=jax.ShapeDtypeStruct((batch, ncls), jnp.float32),
    )
    return fc(pooled, wt, b2)


# speedup vs baseline: 26.9164x; 26.9164x over previous
import functools

import jax
import jax.numpy as jnp
from jax import lax
from jax.experimental import pallas as pl
from jax.experimental.pallas import tpu as pltpu
from jax.experimental.pallas import tpu_sc as plsc


def _transpose_body(bt, d, tab_t_ref, out_ref):
    x = tab_t_ref[...]
    y = x.T
    out_ref[:, 0:d] = y[0:bt // 2, :]
    out_ref[:, d:2 * d] = y[bt // 2:bt, :]


def _make_relayout(v, d, bt):
    grid = (v + bt - 1) // bt
    return pl.pallas_call(
        functools.partial(_transpose_body, bt, d),
        grid=(grid,),
        in_specs=[pl.BlockSpec((d, bt), lambda i: (0, i))],
        out_specs=pl.BlockSpec((bt // 2, 2 * d), lambda i: (i, 0)),
        out_shape=jax.ShapeDtypeStruct((grid * (bt // 2), 2 * d),
                                       jnp.float32),
    )


def _make_pool(batch, hist, d, nc, ns):
    nw = nc * ns
    hd = d // 2
    ct = 2 * hist
    chunks_per_w = batch // nw
    bags_per_w = batch // nw
    half = chunks_per_w // 2
    assert batch % nw == 0 and chunks_per_w % 2 == 0 and hd % 16 == 0
    nh = hd // 16

    mesh = plsc.VectorSubcoreMesh(core_axis_name="c", subcore_axis_name="s")

    @functools.partial(
        pl.kernel,
        mesh=mesh,
        out_type=jax.ShapeDtypeStruct((batch, d), jnp.float32),
        compiler_params=pltpu.CompilerParams(use_tc_tiling_on_sc=False),
        scratch_types=[
            pltpu.VMEM((chunks_per_w, ct), jnp.int32),
            pltpu.VMEM((ct, hd), jnp.float32),
            pltpu.VMEM((ct, hd), jnp.float32),
            pltpu.VMEM((bags_per_w, d), jnp.float32),
            pltpu.SemaphoreType.DMA,
            pltpu.SemaphoreType.DMA,
        ],
    )
    def pool(tok_hbm, table_hbm, out_hbm, idx_v, buf0, buf1, pooled_v,
             sem0, sem1):
        wid = lax.axis_index("s") * nc + lax.axis_index("c")
        pltpu.sync_copy(tok_hbm.at[pl.ds(wid * chunks_per_w, chunks_per_w)],
                        idx_v)

        def start(c, buf, sem):
            pltpu.make_async_copy(table_hbm.at[idx_v.at[c]], buf, sem).start()

        def wait(buf, sem):
            pltpu.make_async_copy(table_hbm.at[idx_v.at[0]], buf, sem).wait()

        def accum(buf, bag):
            accs = [buf[r, pl.ds(k * 16, 16)]
                    for r in (0, 1) for k in range(nh)]
            for t in range(1, hist):
                for r in (0, 1):
                    for k in range(nh):
                        accs[r * nh + k] = (
                            accs[r * nh + k]
                            + buf[2 * t + r, pl.ds(k * 16, 16)])
            for r in (0, 1):
                for k in range(nh):
                    pooled_v[bag, pl.ds((r * nh + k) * 16, 16)] = (
                        accs[r * nh + k])

        start(0, buf0, sem0)

        def step(g, carry):
            start(2 * g + 1, buf1, sem1)
            wait(buf0, sem0)
            accum(buf0, 2 * g)

            @pl.when(g < half - 1)
            def _():
                start(2 * g + 2, buf0, sem0)

            wait(buf1, sem1)
            accum(buf1, 2 * g + 1)
            return carry

        lax.fori_loop(0, half, step, 0)
        pltpu.sync_copy(pooled_v,
                        out_hbm.at[pl.ds(wid * bags_per_w, bags_per_w)])

    return pool


def _fc_body(scale, pooled_ref, wt_ref, b_ref, out_ref):
    out_ref[...] = (
        jnp.dot(pooled_ref[...] * scale, wt_ref[...],
                preferred_element_type=jnp.float32)
        + b_ref[...]
    )


def kernel(concated_token_lists, offsets, emb_weight, fc_weight, fc_bias):
    total = concated_token_lists.shape[0]
    batch = offsets.shape[0]
    hist = total // batch
    v, d = emb_weight.shape
    ncls = fc_weight.shape[0]

    info = plsc.get_sparse_core_info()
    nc, ns = info.num_cores, info.num_subcores

    bt = 1024
    table_lin = _make_relayout(v, d, bt)(emb_weight.T)
    table2 = table_lin.reshape(-1, d // 2)

    tok = concated_token_lists.astype(jnp.int32).reshape(batch, hist)
    tb = tok // bt
    l = tok % bt
    base = 2 * bt * tb + 4 * (l % (bt // 2)) + 2 * (l // (bt // 2))
    pairs = (base[:, :, None] + jnp.arange(2, dtype=jnp.int32)).reshape(
        batch, 2 * hist)

    pooled = _make_pool(batch, hist, d, nc, ns)(pairs, table2)

    wt = fc_weight.T.astype(jnp.float32)
    b2 = fc_bias.reshape(1, ncls).astype(jnp.float32)
    fc = pl.pallas_call(
        functools.partial(_fc_body, 1.0 / float(hist)),
        out_shape=jax.ShapeDtypeStruct((batch, ncls), jnp.float32),
    )
    return fc(pooled, wt, b2)


# speedup vs baseline: 64.2383x; 2.3866x over previous
import functools

import jax
import jax.numpy as jnp
from jax import lax
from jax.experimental import pallas as pl
from jax.experimental.pallas import tpu as pltpu
from jax.experimental.pallas import tpu_sc as plsc


def _transpose_body(bt, d, tab_t_ref, out_ref):
    x = tab_t_ref[...]
    y = x.T
    out_ref[:, 0:d] = y[0:bt // 2, :]
    out_ref[:, d:2 * d] = y[bt // 2:bt, :]


def _make_relayout(v, d, bt):
    grid = (v + bt - 1) // bt
    return pl.pallas_call(
        functools.partial(_transpose_body, bt, d),
        grid=(grid,),
        in_specs=[pl.BlockSpec((d, bt), lambda i: (0, i))],
        out_specs=pl.BlockSpec((bt // 2, 2 * d), lambda i: (i, 0)),
        out_shape=jax.ShapeDtypeStruct((grid * (bt // 2), 2 * d),
                                       jnp.float32),
    )


def _make_pool(batch, hist, d, nrows, nc, ns):
    nw = nc * ns
    cb = 2
    ct = cb * hist
    chunks_per_w = batch // (cb * nw)
    bags_per_w = batch // nw
    half = chunks_per_w // 2
    assert batch % (cb * nw) == 0 and chunks_per_w % 2 == 0 and d % 16 == 0
    nd = d // 16

    mesh = plsc.VectorSubcoreMesh(core_axis_name="c", subcore_axis_name="s")

    @functools.partial(
        pl.kernel,
        mesh=mesh,
        out_type=jax.ShapeDtypeStruct((batch, d), jnp.float32),
        compiler_params=pltpu.CompilerParams(use_tc_tiling_on_sc=False),
        scratch_types=[
            pltpu.VMEM((chunks_per_w, ct), jnp.int32),
            pltpu.VMEM((ct, d), jnp.float32),
            pltpu.VMEM((ct, d), jnp.float32),
            pltpu.VMEM((bags_per_w, d), jnp.float32),
            pltpu.SemaphoreType.DMA,
            pltpu.SemaphoreType.DMA,
        ],
    )
    def pool(tok_hbm, table_hbm, out_hbm, idx_v, buf0, buf1, pooled_v,
             sem0, sem1):
        wid = lax.axis_index("s") * nc + lax.axis_index("c")
        pltpu.sync_copy(tok_hbm.at[pl.ds(wid * chunks_per_w, chunks_per_w)],
                        idx_v)

        def start(c, buf, sem):
            pltpu.make_async_copy(table_hbm.at[idx_v.at[c]], buf, sem).start()

        def wait(buf, sem):
            pltpu.make_async_copy(table_hbm.at[idx_v.at[0]], buf, sem).wait()

        def accum(buf, bag_base):
            for j in range(cb):
                accs = [buf[j * hist, pl.ds(k * 16, 16)] for k in range(nd)]
                for t in range(1, hist):
                    for k in range(nd):
                        accs[k] = accs[k] + buf[j * hist + t,
                                                pl.ds(k * 16, 16)]
                for k in range(nd):
                    pooled_v[bag_base + j, pl.ds(k * 16, 16)] = accs[k]

        start(0, buf0, sem0)

        def step(g, carry):
            start(2 * g + 1, buf1, sem1)
            wait(buf0, sem0)
            accum(buf0, 2 * cb * g)

            @pl.when(g < half - 1)
            def _():
                start(2 * g + 2, buf0, sem0)

            wait(buf1, sem1)
            accum(buf1, 2 * cb * g + cb)
            return carry

        lax.fori_loop(0, half, step, 0)
        pltpu.sync_copy(pooled_v,
                        out_hbm.at[pl.ds(wid * bags_per_w, bags_per_w)])

    return pool


def _fc_body(scale, pooled_ref, wt_ref, b_ref, out_ref):
    out_ref[...] = (
        jnp.dot(pooled_ref[...] * scale, wt_ref[...],
                preferred_element_type=jnp.float32)
        + b_ref[...]
    )


def kernel(concated_token_lists, offsets, emb_weight, fc_weight, fc_bias):
    total = concated_token_lists.shape[0]
    batch = offsets.shape[0]
    hist = total // batch
    v, d = emb_weight.shape
    ncls = fc_weight.shape[0]

    info = plsc.get_sparse_core_info()
    nc, ns = info.num_cores, info.num_subcores

    bt = 8192
    table_lin = _make_relayout(v, d, bt)(emb_weight.T)
    table64 = table_lin.reshape(-1, d)
    nrows = table64.shape[0]

    cb = 2
    tok = concated_token_lists.astype(jnp.int32)
    l = tok % bt
    rmap = bt * (tok // bt) + 2 * (l % (bt // 2)) + l // (bt // 2)
    tok2d = rmap.reshape(batch // cb, hist * cb)

    pooled = _make_pool(batch, hist, d, nrows, nc, ns)(tok2d, table64)

    wt = fc_weight.T.astype(jnp.float32)
    b2 = fc_bias.reshape(1, ncls).astype(jnp.float32)
    fc = pl.pallas_call(
        functools.partial(_fc_body, 1.0 / float(hist)),
        out_shape=jax.ShapeDtypeStruct((batch, ncls), jnp.float32),
    )
    return fc(pooled, wt, b2)


# speedup vs baseline: 71.3441x; 1.1106x over previous
import functools

import jax
import jax.numpy as jnp
from jax import lax
from jax.experimental import pallas as pl
from jax.experimental.pallas import tpu as pltpu
from jax.experimental.pallas import tpu_sc as plsc


def _transpose_body(bt, d, tab_t_ref, out_ref):
    x = tab_t_ref[...]
    y = x.T
    out_ref[:, 0:d] = y[0:bt // 2, :]
    out_ref[:, d:2 * d] = y[bt // 2:bt, :]


def _make_relayout(v, d, bt):
    grid = (v + bt - 1) // bt
    return pl.pallas_call(
        functools.partial(_transpose_body, bt, d),
        grid=(grid,),
        in_specs=[pl.BlockSpec((d, bt), lambda i: (0, i))],
        out_specs=pl.BlockSpec((bt // 2, 2 * d), lambda i: (i, 0)),
        out_shape=jax.ShapeDtypeStruct((grid * (bt // 2), 2 * d),
                                       jnp.float32),
    )


def _make_pool(batch, hist, d, nrows, nc, ns):
    nw = nc * ns
    cb = 2
    ct = cb * hist
    chunks_per_w = batch // (cb * nw)
    bags_per_w = batch // nw
    half = chunks_per_w // 2
    assert batch % (cb * nw) == 0 and chunks_per_w % 2 == 0 and d % 16 == 0
    nd = d // 16

    mesh = plsc.VectorSubcoreMesh(core_axis_name="c", subcore_axis_name="s")

    @functools.partial(
        pl.kernel,
        mesh=mesh,
        out_type=jax.ShapeDtypeStruct((batch, d), jnp.float32),
        compiler_params=pltpu.CompilerParams(use_tc_tiling_on_sc=False),
        scratch_types=[
            pltpu.VMEM((chunks_per_w, ct), jnp.int32),
            pltpu.VMEM((ct, d), jnp.float32),
            pltpu.VMEM((ct, d), jnp.float32),
            pltpu.VMEM((bags_per_w, d), jnp.float32),
            pltpu.SemaphoreType.DMA,
            pltpu.SemaphoreType.DMA,
        ],
    )
    def pool(tok_hbm, table_hbm, out_hbm, idx_v, buf0, buf1, pooled_v,
             sem0, sem1):
        wid = lax.axis_index("s") * nc + lax.axis_index("c")
        pltpu.sync_copy(tok_hbm.at[pl.ds(wid * chunks_per_w, chunks_per_w)],
                        idx_v)

        def start(c, buf, sem):
            pltpu.make_async_copy(table_hbm.at[idx_v.at[c]], buf, sem).start()

        def wait(buf, sem):
            pltpu.make_async_copy(table_hbm.at[idx_v.at[0]], buf, sem).wait()

        def accum(buf, bag_base):
            for j in range(cb):
                accs = [buf[j * hist, pl.ds(k * 16, 16)] for k in range(nd)]
                for t in range(1, hist):
                    for k in range(nd):
                        accs[k] = accs[k] + buf[j * hist + t,
                                                pl.ds(k * 16, 16)]
                for k in range(nd):
                    pooled_v[bag_base + j, pl.ds(k * 16, 16)] = accs[k]

        start(0, buf0, sem0)

        def step(g, carry):
            start(2 * g + 1, buf1, sem1)
            wait(buf0, sem0)
            accum(buf0, 2 * cb * g)

            @pl.when(g < half - 1)
            def _():
                start(2 * g + 2, buf0, sem0)

            wait(buf1, sem1)
            accum(buf1, 2 * cb * g + cb)
            return carry

        lax.fori_loop(0, half, step, 0)
        pltpu.sync_copy(pooled_v,
                        out_hbm.at[pl.ds(wid * bags_per_w, bags_per_w)])

    return pool


def _fc_body(scale, pooled_ref, wt_ref, b_ref, out_ref):
    out_ref[...] = (
        jnp.dot(pooled_ref[...] * scale, wt_ref[...],
                preferred_element_type=jnp.float32)
        + b_ref[...]
    )


def kernel(concated_token_lists, offsets, emb_weight, fc_weight, fc_bias):
    total = concated_token_lists.shape[0]
    batch = offsets.shape[0]
    hist = total // batch
    v, d = emb_weight.shape
    ncls = fc_weight.shape[0]

    info = plsc.get_sparse_core_info()
    nc, ns = info.num_cores, info.num_subcores

    bt = 16384
    table_lin = _make_relayout(v, d, bt)(emb_weight.T)
    table64 = table_lin.reshape(-1, d)
    nrows = table64.shape[0]

    cb = 2
    tok = concated_token_lists.astype(jnp.int32)
    l = tok % bt
    rmap = bt * (tok // bt) + 2 * (l % (bt // 2)) + l // (bt // 2)
    tok2d = rmap.reshape(batch // cb, hist * cb)

    pooled = _make_pool(batch, hist, d, nrows, nc, ns)(tok2d, table64)

    wt = fc_weight.T.astype(jnp.float32)
    b2 = fc_bias.reshape(1, ncls).astype(jnp.float32)
    fc = pl.pallas_call(
        functools.partial(_fc_body, 1.0 / float(hist)),
        out_shape=jax.ShapeDtypeStruct((batch, ncls), jnp.float32),
    )
    return fc(pooled, wt, b2)


# speedup vs baseline: 75.0820x; 1.0524x over previous
import functools

import jax
import jax.numpy as jnp
from jax import lax
from jax.experimental import pallas as pl
from jax.experimental.pallas import tpu as pltpu
from jax.experimental.pallas import tpu_sc as plsc


def _transpose_body(bt, d, tab_t_ref, out_ref):
    x = tab_t_ref[...]
    y = x.T
    out_ref[:, 0:d] = y[0:bt // 2, :]
    out_ref[:, d:2 * d] = y[bt // 2:bt, :]


def _make_relayout(v, d, bt):
    grid = (v + bt - 1) // bt
    return pl.pallas_call(
        functools.partial(_transpose_body, bt, d),
        grid=(grid,),
        in_specs=[pl.BlockSpec((d, bt), lambda i: (0, i))],
        out_specs=pl.BlockSpec((bt // 2, 2 * d), lambda i: (i, 0)),
        out_shape=jax.ShapeDtypeStruct((grid * (bt // 2), 2 * d),
                                       jnp.float32),
    )


def _make_pool(batch, hist, d, nrows, nc, ns):
    nw = nc * ns
    cb = 2
    ct = cb * hist
    chunks_per_w = batch // (cb * nw)
    bags_per_w = batch // nw
    half = chunks_per_w // 2
    assert batch % (cb * nw) == 0 and chunks_per_w % 2 == 0 and d % 16 == 0
    nd = d // 16

    mesh = plsc.VectorSubcoreMesh(core_axis_name="c", subcore_axis_name="s")

    @functools.partial(
        pl.kernel,
        mesh=mesh,
        out_type=jax.ShapeDtypeStruct((batch, d), jnp.float32),
        compiler_params=pltpu.CompilerParams(use_tc_tiling_on_sc=False),
        scratch_types=[
            pltpu.VMEM((chunks_per_w, ct), jnp.int32),
            pltpu.VMEM((ct, d), jnp.float32),
            pltpu.VMEM((ct, d), jnp.float32),
            pltpu.VMEM((bags_per_w, d), jnp.float32),
            pltpu.SemaphoreType.DMA,
            pltpu.SemaphoreType.DMA,
        ],
    )
    def pool(tok_hbm, table_hbm, out_hbm, idx_v, buf0, buf1, pooled_v,
             sem0, sem1):
        wid = lax.axis_index("s") * nc + lax.axis_index("c")
        pltpu.sync_copy(tok_hbm.at[pl.ds(wid * chunks_per_w, chunks_per_w)],
                        idx_v)

        def start(c, buf, sem):
            pltpu.make_async_copy(table_hbm.at[idx_v.at[c]], buf, sem).start()

        def wait(buf, sem):
            pltpu.make_async_copy(table_hbm.at[idx_v.at[0]], buf, sem).wait()

        def accum(buf, bag_base):
            for j in range(cb):
                accs = [buf[j * hist, pl.ds(k * 16, 16)] for k in range(nd)]
                for t in range(1, hist):
                    for k in range(nd):
                        accs[k] = accs[k] + buf[j * hist + t,
                                                pl.ds(k * 16, 16)]
                for k in range(nd):
                    pooled_v[bag_base + j, pl.ds(k * 16, 16)] = accs[k]

        start(0, buf0, sem0)

        def step(g, carry):
            start(2 * g + 1, buf1, sem1)
            wait(buf0, sem0)
            accum(buf0, 2 * cb * g)

            @pl.when(g < half - 1)
            def _():
                start(2 * g + 2, buf0, sem0)

            wait(buf1, sem1)
            accum(buf1, 2 * cb * g + cb)
            return carry

        lax.fori_loop(0, half, step, 0)
        pltpu.sync_copy(pooled_v,
                        out_hbm.at[pl.ds(wid * bags_per_w, bags_per_w)])

    return pool


def _fc_body(scale, pooled_ref, wt_ref, b_ref, out_ref):
    out_ref[...] = (
        jnp.dot(pooled_ref[...] * scale, wt_ref[...],
                preferred_element_type=jnp.float32)
        + b_ref[...]
    )


def kernel(concated_token_lists, offsets, emb_weight, fc_weight, fc_bias):
    total = concated_token_lists.shape[0]
    batch = offsets.shape[0]
    hist = total // batch
    v, d = emb_weight.shape
    ncls = fc_weight.shape[0]

    info = plsc.get_sparse_core_info()
    nc, ns = info.num_cores, info.num_subcores

    bt = 32768
    table_lin = _make_relayout(v, d, bt)(emb_weight.T)
    table64 = table_lin.reshape(-1, d)
    nrows = table64.shape[0]

    cb = 2
    tok = concated_token_lists.astype(jnp.int32)
    l = tok % bt
    rmap = bt * (tok // bt) + 2 * (l % (bt // 2)) + l // (bt // 2)
    tok2d = rmap.reshape(batch // cb, hist * cb)

    pooled = _make_pool(batch, hist, d, nrows, nc, ns)(tok2d, table64)

    wt = fc_weight.T.astype(jnp.float32)
    b2 = fc_bias.reshape(1, ncls).astype(jnp.float32)
    fc = pl.pallas_call(
        functools.partial(_fc_body, 1.0 / float(hist)),
        out_shape=jax.ShapeDtypeStruct((batch, ncls), jnp.float32),
    )
    return fc(pooled, wt, b2)
